# Initial kernel scaffold; baseline (speedup 1.0000x reference)
import jax
import jax.numpy as jnp
from jax.experimental import pallas as pl


def kernel(x, gate_w1, gate_w2, W_gate, W_up, W_down, b_gate, b_up, b_down):
    raise NotImplementedError("write your pallas kernel here")


# speedup vs baseline: 2.5089x; 2.5089x over previous
import functools

import jax
import jax.numpy as jnp
from jax.experimental import pallas as pl
from jax.experimental.pallas import tpu as pltpu

INPUT_SIZE = 1024
HIDDEN_SIZE = 176
OUTPUT_SIZE = 1024
NUM_EXPERTS = 64
NUM_SELECTS = 2
BALANCE_LOSS_WEIGHT = 1e-2

_GATE_BLK = 512
_EXP_BLK = 2048


def _gate_body(x_ref, g1_ref, g2_ref, sf_ref, loss_ref, imp_ref, load_ref):
    tb = pl.program_id(0)
    nb = pl.num_programs(0)
    x = x_ref[...]
    t1 = jnp.tanh(jax.lax.dot_general(x, g1_ref[...], (((1,), (0,)), ((), ()))))
    logits = jax.lax.dot_general(t1, g2_ref[...], (((1,), (0,)), ((), ())))
    lane = jax.lax.broadcasted_iota(jnp.int32, logits.shape, 1)
    m1 = jnp.max(logits, axis=1, keepdims=True)
    i1 = jnp.min(jnp.where(logits == m1, lane, NUM_EXPERTS), axis=1, keepdims=True)
    mask1 = lane == i1
    l2 = jnp.where(mask1, -jnp.inf, logits)
    m2 = jnp.max(l2, axis=1, keepdims=True)
    i2 = jnp.min(jnp.where(l2 == m2, lane, NUM_EXPERTS), axis=1, keepdims=True)
    mask2 = lane == i2
    s1 = 1.0 / (1.0 + jnp.exp(m2 - m1))
    sf = jnp.where(mask1, s1, 0.0) + jnp.where(mask2, 1.0 - s1, 0.0)
    sf_ref[...] = sf

    @pl.when(tb == 0)
    def _init():
        imp_ref[...] = jnp.zeros_like(imp_ref)
        load_ref[...] = jnp.zeros_like(load_ref)

    imp_ref[...] += jnp.sum(sf, axis=0, keepdims=True)
    load_ref[...] += jnp.sum((sf > 0.0).astype(jnp.float32), axis=0, keepdims=True)

    @pl.when(tb == nb - 1)
    def _fin():
        def cv2(v):
            mean = jnp.mean(v)
            var = jnp.sum((v - mean) ** 2) / (NUM_EXPERTS - 1)
            return var / (mean * mean + 1e-10)

        loss = BALANCE_LOSS_WEIGHT * (cv2(imp_ref[...]) + cv2(load_ref[...]))
        loss_ref[...] = loss * jnp.ones((1, 1), jnp.float32)


def _expert_body(x_ref, wg_ref, wu_ref, wd_ref, bg_ref, bu_ref, bd_ref, sf_ref,
                 y_ref):
    e = pl.program_id(1)
    x = x_ref[...]
    g = jax.lax.dot_general(x, wg_ref[0], (((1,), (1,)), ((), ()))) + bg_ref[0]
    u = jax.lax.dot_general(x, wu_ref[0], (((1,), (1,)), ((), ()))) + bu_ref[0]
    h = (g / (1.0 + jnp.exp(-g))) * u
    out = jax.lax.dot_general(h, wd_ref[0], (((1,), (1,)), ((), ()))) + bd_ref[0]
    sf = sf_ref[...]
    lane = jax.lax.broadcasted_iota(jnp.int32, sf.shape, 1)
    w = jnp.sum(jnp.where(lane == e, sf, 0.0), axis=1, keepdims=True)

    @pl.when(e == 0)
    def _init():
        y_ref[...] = w * out

    @pl.when(e != 0)
    def _acc():
        y_ref[...] += w * out


def kernel(x, gate_w1, gate_w2, W_gate, W_up, W_down, b_gate, b_up, b_down):
    B, S, _ = x.shape
    T = B * S
    xf = x.reshape(T, INPUT_SIZE)

    nb = T // _GATE_BLK
    sf, loss = pl.pallas_call(
        _gate_body,
        grid=(nb,),
        in_specs=[
            pl.BlockSpec((_GATE_BLK, INPUT_SIZE), lambda i: (i, 0)),
            pl.BlockSpec((INPUT_SIZE, NUM_EXPERTS), lambda i: (0, 0)),
            pl.BlockSpec((NUM_EXPERTS, NUM_EXPERTS), lambda i: (0, 0)),
        ],
        out_specs=[
            pl.BlockSpec((_GATE_BLK, NUM_EXPERTS), lambda i: (i, 0)),
            pl.BlockSpec((1, 1), lambda i: (0, 0)),
        ],
        out_shape=[
            jax.ShapeDtypeStruct((T, NUM_EXPERTS), jnp.float32),
            jax.ShapeDtypeStruct((1, 1), jnp.float32),
        ],
        scratch_shapes=[
            pltpu.VMEM((1, NUM_EXPERTS), jnp.float32),
            pltpu.VMEM((1, NUM_EXPERTS), jnp.float32),
        ],
    )(xf, gate_w1, gate_w2)

    nc = T // _EXP_BLK
    y = pl.pallas_call(
        _expert_body,
        grid=(nc, NUM_EXPERTS),
        in_specs=[
            pl.BlockSpec((_EXP_BLK, INPUT_SIZE), lambda i, e: (i, 0)),
            pl.BlockSpec((1, HIDDEN_SIZE, INPUT_SIZE), lambda i, e: (e, 0, 0)),
            pl.BlockSpec((1, HIDDEN_SIZE, INPUT_SIZE), lambda i, e: (e, 0, 0)),
            pl.BlockSpec((1, OUTPUT_SIZE, HIDDEN_SIZE), lambda i, e: (e, 0, 0)),
            pl.BlockSpec((1, 1, HIDDEN_SIZE), lambda i, e: (e, 0, 0)),
            pl.BlockSpec((1, 1, HIDDEN_SIZE), lambda i, e: (e, 0, 0)),
            pl.BlockSpec((1, 1, OUTPUT_SIZE), lambda i, e: (e, 0, 0)),
            pl.BlockSpec((_EXP_BLK, NUM_EXPERTS), lambda i, e: (i, 0)),
        ],
        out_specs=pl.BlockSpec((_EXP_BLK, OUTPUT_SIZE), lambda i, e: (i, 0)),
        out_shape=jax.ShapeDtypeStruct((T, OUTPUT_SIZE), jnp.float32),
    )(xf, W_gate, W_up, W_down, b_gate[:, None, :], b_up[:, None, :],
      b_down[:, None, :], sf)

    return y.reshape(B, S, OUTPUT_SIZE), loss.reshape(())


# speedup vs baseline: 5.3162x; 2.1190x over previous
import functools

import jax
import jax.numpy as jnp
from jax.experimental import pallas as pl
from jax.experimental.pallas import tpu as pltpu

INPUT_SIZE = 1024
HIDDEN_SIZE = 176
OUTPUT_SIZE = 1024
NUM_EXPERTS = 64
NUM_SELECTS = 2
BALANCE_LOSS_WEIGHT = 1e-2

_GATE_BLK = 512
_BR = 256


def _gate_body(x_ref, g1_ref, g2_ref, idx_ref, sc_ref, loss_ref, imp_ref,
               load_ref):
    tb = pl.program_id(0)
    nb = pl.num_programs(0)
    x = x_ref[...]
    t1 = jnp.tanh(jax.lax.dot_general(x, g1_ref[...], (((1,), (0,)), ((), ()))))
    logits = jax.lax.dot_general(t1, g2_ref[...], (((1,), (0,)), ((), ())))
    lane = jax.lax.broadcasted_iota(jnp.int32, logits.shape, 1)
    m1 = jnp.max(logits, axis=1, keepdims=True)
    i1 = jnp.min(jnp.where(logits == m1, lane, NUM_EXPERTS), axis=1, keepdims=True)
    mask1 = lane == i1
    l2 = jnp.where(mask1, -jnp.inf, logits)
    m2 = jnp.max(l2, axis=1, keepdims=True)
    i2 = jnp.min(jnp.where(l2 == m2, lane, NUM_EXPERTS), axis=1, keepdims=True)
    mask2 = lane == i2
    s1 = 1.0 / (1.0 + jnp.exp(m2 - m1))
    sf1 = jnp.where(mask1, s1, 0.0)
    sf2 = jnp.where(mask2, 1.0 - s1, 0.0)
    idx_ref[...] = jnp.concatenate([i1, i2], axis=1)
    sc_ref[...] = jnp.concatenate([s1, 1.0 - s1], axis=1)

    @pl.when(tb == 0)
    def _init():
        imp_ref[...] = jnp.zeros_like(imp_ref)
        load_ref[...] = jnp.zeros_like(load_ref)

    imp_ref[...] += jnp.sum(sf1 + sf2, axis=0, keepdims=True)
    load_ref[...] += (jnp.sum((sf1 > 0.0).astype(jnp.float32), axis=0, keepdims=True)
                      + jnp.sum((sf2 > 0.0).astype(jnp.float32), axis=0, keepdims=True))

    @pl.when(tb == nb - 1)
    def _fin():
        def cv2(v):
            mean = jnp.mean(v)
            var = jnp.sum((v - mean) ** 2) / (NUM_EXPERTS - 1)
            return var / (mean * mean + 1e-10)

        loss = BALANCE_LOSS_WEIGHT * (cv2(imp_ref[...]) + cv2(load_ref[...]))
        loss_ref[...] = loss * jnp.ones((1, 1), jnp.float32)


def _group_body(tb, te, tf, rl, rh, xs_ref, ss_ref, wg_ref, wu_ref, wd_ref,
                bg_ref, bu_ref, bd_ref, out_ref):
    i = pl.program_id(0)
    x = xs_ref[...]
    g = jax.lax.dot_general(x, wg_ref[0], (((1,), (1,)), ((), ()))) + bg_ref[0]
    u = jax.lax.dot_general(x, wu_ref[0], (((1,), (1,)), ((), ()))) + bu_ref[0]
    h = (g / (1.0 + jnp.exp(-g))) * u
    out = jax.lax.dot_general(h, wd_ref[0], (((1,), (1,)), ((), ()))) + bd_ref[0]
    rows = tb[i] * _BR + jax.lax.broadcasted_iota(jnp.int32, (_BR, 1), 0)
    m = ((rows >= rl[i]) & (rows < rh[i])).astype(jnp.float32)
    w = ss_ref[...] * m
    contrib = w * out

    @pl.when(tf[i] == 1)
    def _set():
        out_ref[...] = contrib

    @pl.when(tf[i] == 0)
    def _acc():
        out_ref[...] += contrib


def _routing_metadata(idx, sc):
    A = idx.size
    e_flat = idx.reshape(-1)
    s_flat = sc.reshape(-1)
    order = jnp.argsort(e_flat)
    e_sorted = jnp.take(e_flat, order)
    t_sorted = order // NUM_SELECTS
    s_sorted = jnp.take(s_flat, order)
    inv = jnp.argsort(order)
    posA = inv[0::NUM_SELECTS]
    posB = inv[1::NUM_SELECTS]
    ear = jnp.arange(NUM_EXPERTS, dtype=e_sorted.dtype)
    starts = jnp.searchsorted(e_sorted, ear).astype(jnp.int32)
    ends = jnp.searchsorted(e_sorted, ear, side="right").astype(jnp.int32)
    NB = A // _BR
    eb_first = e_sorted[0::_BR].astype(jnp.int32)
    eb_last = e_sorted[_BR - 1::_BR].astype(jnp.int32)
    nb = eb_last - eb_first + 1
    cum = jnp.concatenate([jnp.zeros(1, jnp.int32), jnp.cumsum(nb)]).astype(jnp.int32)
    total = cum[NB]
    NT = NB + NUM_EXPERTS
    ti = jnp.arange(NT, dtype=jnp.int32)
    blk = jnp.clip(jnp.searchsorted(cum, ti, side="right") - 1, 0, NB - 1).astype(jnp.int32)
    pad = ti >= total
    tile_block = jnp.where(pad, NB - 1, blk)
    tile_expert = jnp.where(pad, eb_last[NB - 1],
                            eb_first[blk] + (ti - cum[blk]))
    row_lo = jnp.where(pad, 0, jnp.maximum(starts[tile_expert], tile_block * _BR))
    row_hi = jnp.where(pad, 0, jnp.minimum(ends[tile_expert], (tile_block + 1) * _BR))
    tile_first = ((ti == cum[blk]) & ~pad).astype(jnp.int32)
    return (order, t_sorted, s_sorted, posA, posB, tile_block, tile_expert,
            tile_first, row_lo, row_hi)


def kernel(x, gate_w1, gate_w2, W_gate, W_up, W_down, b_gate, b_up, b_down):
    B, S, _ = x.shape
    T = B * S
    A = T * NUM_SELECTS
    xf = x.reshape(T, INPUT_SIZE)

    nbg = T // _GATE_BLK
    idx, sc, loss = pl.pallas_call(
        _gate_body,
        grid=(nbg,),
        in_specs=[
            pl.BlockSpec((_GATE_BLK, INPUT_SIZE), lambda i: (i, 0)),
            pl.BlockSpec((INPUT_SIZE, NUM_EXPERTS), lambda i: (0, 0)),
            pl.BlockSpec((NUM_EXPERTS, NUM_EXPERTS), lambda i: (0, 0)),
        ],
        out_specs=[
            pl.BlockSpec((_GATE_BLK, NUM_SELECTS), lambda i: (i, 0)),
            pl.BlockSpec((_GATE_BLK, NUM_SELECTS), lambda i: (i, 0)),
            pl.BlockSpec((1, 1), lambda i: (0, 0)),
        ],
        out_shape=[
            jax.ShapeDtypeStruct((T, NUM_SELECTS), jnp.int32),
            jax.ShapeDtypeStruct((T, NUM_SELECTS), jnp.float32),
            jax.ShapeDtypeStruct((1, 1), jnp.float32),
        ],
        scratch_shapes=[
            pltpu.VMEM((1, NUM_EXPERTS), jnp.float32),
            pltpu.VMEM((1, NUM_EXPERTS), jnp.float32),
        ],
    )(xf, gate_w1, gate_w2)

    (order, t_sorted, s_sorted, posA, posB, tile_block, tile_expert,
     tile_first, row_lo, row_hi) = _routing_metadata(idx, sc)

    xs = jnp.take(xf, t_sorted, axis=0)

    NT = A // _BR + NUM_EXPERTS
    outs = pl.pallas_call(
        _group_body,
        grid_spec=pltpu.PrefetchScalarGridSpec(
            num_scalar_prefetch=5,
            grid=(NT,),
            in_specs=[
                pl.BlockSpec((_BR, INPUT_SIZE),
                             lambda i, tb, te, tf, rl, rh: (tb[i], 0)),
                pl.BlockSpec((_BR, 1),
                             lambda i, tb, te, tf, rl, rh: (tb[i], 0)),
                pl.BlockSpec((1, HIDDEN_SIZE, INPUT_SIZE),
                             lambda i, tb, te, tf, rl, rh: (te[i], 0, 0)),
                pl.BlockSpec((1, HIDDEN_SIZE, INPUT_SIZE),
                             lambda i, tb, te, tf, rl, rh: (te[i], 0, 0)),
                pl.BlockSpec((1, OUTPUT_SIZE, HIDDEN_SIZE),
                             lambda i, tb, te, tf, rl, rh: (te[i], 0, 0)),
                pl.BlockSpec((1, 1, HIDDEN_SIZE),
                             lambda i, tb, te, tf, rl, rh: (te[i], 0, 0)),
                pl.BlockSpec((1, 1, HIDDEN_SIZE),
                             lambda i, tb, te, tf, rl, rh: (te[i], 0, 0)),
                pl.BlockSpec((1, 1, OUTPUT_SIZE),
                             lambda i, tb, te, tf, rl, rh: (te[i], 0, 0)),
            ],
            out_specs=pl.BlockSpec((_BR, OUTPUT_SIZE),
                                   lambda i, tb, te, tf, rl, rh: (tb[i], 0)),
        ),
        out_shape=jax.ShapeDtypeStruct((A, OUTPUT_SIZE), jnp.float32),
    )(tile_block, tile_expert, tile_first, row_lo, row_hi,
      xs, s_sorted.reshape(A, 1), W_gate, W_up, W_down,
      b_gate[:, None, :], b_up[:, None, :], b_down[:, None, :])

    y = jnp.take(outs, posA, axis=0) + jnp.take(outs, posB, axis=0)

    return y.reshape(B, S, OUTPUT_SIZE), loss.reshape(())
